# baseline (speedup 1.0000x reference)
import math

import jax
import jax.numpy as jnp
from jax.experimental import pallas as pl
from jax.experimental.pallas import tpu as pltpu

_B, _N, _D, _E, _KE = 2, 2048, 1024, 16, 8
_NT = 512

_DN0 = (((0,), (0,)), ((), ()))


def _assign_tile_t(idxT, wT):
    nt = idxT.shape[-1]
    iota_e = jax.lax.broadcasted_iota(jnp.int32, (_E, nt), 0)
    acc = jnp.zeros((_E, nt), jnp.float32)
    for k in range(_KE):
        acc = acc + jnp.where(idxT[k:k + 1, :] == iota_e, wT[k:k + 1, :], 0.0)
    return acc


def _gelu_exact(x):
    return 0.5 * x * (1.0 + jax.lax.erf(x * (1.0 / math.sqrt(2.0))))


def _he_body(idxAT_ref, wAT_ref, idxBT_ref, wBT_ref, xA_ref, xB_ref,
             heA_ref, heB_ref):
    nt = pl.program_id(1)
    At = _assign_tile_t(idxAT_ref[0], wAT_ref[0])
    Bt = _assign_tile_t(idxBT_ref[0], wBT_ref[0])
    heA = jnp.dot(At, xA_ref[0], preferred_element_type=jnp.float32)
    heB = jnp.dot(Bt, xB_ref[0], preferred_element_type=jnp.float32)

    @pl.when(nt == 0)
    def _():
        heA_ref[0] = heA
        heB_ref[0] = heB

    @pl.when(nt != 0)
    def _():
        heA_ref[0] += heA
        heB_ref[0] += heB


def _main_body(idxAT_ref, wAT_ref, idxBT_ref, wBT_ref, xA_ref, xB_ref,
               heA_ref, heB_ref, wb2a_ref, bb2a_ref, wa2b_ref, ba2b_ref,
               wgA_top_ref, wgA_bot_ref, wgB_top_ref, wgB_bot_ref,
               bgA_ref, bgB_ref,
               outA_ref, outB_ref,
               heAt_s, heBt_s, mA_s, mB_s):
    @pl.when(pl.program_id(1) == 0)
    def _():
        heAt = _gelu_exact(
            jnp.dot(heA_ref[0], wb2a_ref[...],
                    preferred_element_type=jnp.float32) + bb2a_ref[...])
        heBt = _gelu_exact(
            jnp.dot(heB_ref[0], wa2b_ref[...],
                    preferred_element_type=jnp.float32) + ba2b_ref[...])
        heAt_s[...] = heAt
        heBt_s[...] = heBt
        mA_s[...] = jnp.dot(heBt, wgA_bot_ref[...],
                            preferred_element_type=jnp.float32)
        mB_s[...] = jnp.dot(heAt, wgB_bot_ref[...],
                            preferred_element_type=jnp.float32)

    At = _assign_tile_t(idxAT_ref[0], wAT_ref[0])
    Bt = _assign_tile_t(idxBT_ref[0], wBT_ref[0])

    xA = xA_ref[0]
    preA = (jnp.dot(xA.astype(jnp.bfloat16), wgA_top_ref[...],
                    preferred_element_type=jnp.float32)
            + jax.lax.dot_general(At, mA_s[...], _DN0,
                                  preferred_element_type=jnp.float32)
            + bgA_ref[...])
    gA = jax.nn.sigmoid(preA)
    xAfromB = jax.lax.dot_general(At, heBt_s[...], _DN0,
                                  preferred_element_type=jnp.float32)
    outA_ref[0] = gA * xA + (1.0 - gA) * xAfromB

    xB = xB_ref[0]
    preB = (jnp.dot(xB.astype(jnp.bfloat16), wgB_top_ref[...],
                    preferred_element_type=jnp.float32)
            + jax.lax.dot_general(Bt, mB_s[...], _DN0,
                                  preferred_element_type=jnp.float32)
            + bgB_ref[...])
    gB = jax.nn.sigmoid(preB)
    xBfromA = jax.lax.dot_general(Bt, heAt_s[...], _DN0,
                                  preferred_element_type=jnp.float32)
    outB_ref[0] = gB * xB + (1.0 - gB) * xBfromA


def kernel(X_A, X_B, idxA, wA, idxB, wB, E, W_A2B, b_A2B, W_B2A, b_B2A,
           Wg_A, bg_A, Wg_B, bg_B):
    del E
    f32 = jnp.float32
    nnt = _N // _NT

    idxAT = jnp.swapaxes(idxA, 1, 2)
    wAT = jnp.swapaxes(wA, 1, 2)
    idxBT = jnp.swapaxes(idxB, 1, 2)
    wBT = jnp.swapaxes(wB, 1, 2)

    idxt_spec = pl.BlockSpec((1, _KE, _NT), lambda b, n: (b, 0, n))
    x_spec = pl.BlockSpec((1, _NT, _D), lambda b, n: (b, n, 0))
    he_spec = pl.BlockSpec((1, _E, _D), lambda b, n: (b, 0, 0))

    he_A, he_B = pl.pallas_call(
        _he_body,
        grid=(_B, nnt),
        in_specs=[idxt_spec, idxt_spec, idxt_spec, idxt_spec, x_spec, x_spec],
        out_specs=[he_spec, he_spec],
        out_shape=[jax.ShapeDtypeStruct((_B, _E, _D), f32),
                   jax.ShapeDtypeStruct((_B, _E, _D), f32)],
    )(idxAT, wAT, idxBT, wBT, X_A, X_B)

    wgA_top = Wg_A[:_D].astype(jnp.bfloat16)
    wgB_top = Wg_B[:_D].astype(jnp.bfloat16)

    w_spec = pl.BlockSpec((_D, _D), lambda b, n: (0, 0))
    wg_top_spec = pl.BlockSpec((_D, _D), lambda b, n: (0, 0))
    wg_bot_spec = pl.BlockSpec((_D, _D), lambda b, n: (1, 0))
    bias_spec = pl.BlockSpec((1, _D), lambda b, n: (0, 0))
    scr = pltpu.VMEM((_E, _D), f32)

    out_A, out_B = pl.pallas_call(
        _main_body,
        grid=(_B, nnt),
        in_specs=[idxt_spec, idxt_spec, idxt_spec, idxt_spec, x_spec, x_spec,
                  he_spec, he_spec,
                  w_spec, bias_spec, w_spec, bias_spec,
                  wg_top_spec, wg_bot_spec, wg_top_spec, wg_bot_spec,
                  bias_spec, bias_spec],
        out_specs=[x_spec, x_spec],
        out_shape=[jax.ShapeDtypeStruct((_B, _N, _D), f32),
                   jax.ShapeDtypeStruct((_B, _N, _D), f32)],
        scratch_shapes=[scr, scr, scr, scr],
    )(idxAT, wAT, idxBT, wBT, X_A, X_B, he_A, he_B,
      W_B2A, b_B2A.reshape(1, _D), W_A2B, b_A2B.reshape(1, _D),
      wgA_top, Wg_A, wgB_top, Wg_B,
      bg_A.reshape(1, _D), bg_B.reshape(1, _D))

    return (out_A, out_B)


# speedup vs baseline: 1.1307x; 1.1307x over previous
import math

import jax
import jax.numpy as jnp
from jax.experimental import pallas as pl
from jax.experimental.pallas import tpu as pltpu

_B, _N, _D, _E, _KE = 2, 2048, 1024, 16, 8
_NT = 512
_NNT = _N // _NT

_DN0 = (((0,), (0,)), ((), ()))


def _assign_tile_t(idxT, wT):
    nt = idxT.shape[-1]
    iota_e = jax.lax.broadcasted_iota(jnp.int32, (_E, nt), 0)
    acc = jnp.zeros((_E, nt), jnp.float32)
    for k in range(_KE):
        acc = acc + jnp.where(idxT[k:k + 1, :] == iota_e, wT[k:k + 1, :], 0.0)
    return acc


def _gelu_exact(x):
    return 0.5 * x * (1.0 + jax.lax.erf(x * (1.0 / math.sqrt(2.0))))


def _body(idxAT_ref, wAT_ref, idxBT_ref, wBT_ref, xA_ref, xB_ref,
          wb2a_ref, bb2a_ref, wa2b_ref, ba2b_ref,
          wgA_ref, wgB_ref, bgA_ref, bgB_ref,
          outA_ref, outB_ref,
          xAs, xBs, AtS, BtS, heA_s, heB_s,
          heAt_s, heBt_s, mA_s, mB_s, wgAtop_s, wgBtop_s):
    b = pl.program_id(0)
    ph = pl.program_id(1)
    nt = pl.program_id(2)
    nsl = pl.ds(nt * _NT, _NT)

    @pl.when(jnp.logical_and(jnp.logical_and(b == 0, ph == 0), nt == 0))
    def _():
        wgAtop_s[...] = wgA_ref[:_D, :].astype(jnp.bfloat16)
        wgBtop_s[...] = wgB_ref[:_D, :].astype(jnp.bfloat16)

    @pl.when(ph == 0)
    def _():
        At = _assign_tile_t(idxAT_ref[0], wAT_ref[0])
        Bt = _assign_tile_t(idxBT_ref[0], wBT_ref[0])
        AtS[:, nsl] = At
        BtS[:, nsl] = Bt
        xa = xA_ref[0]
        xb = xB_ref[0]
        xAs[nsl, :] = xa.astype(jnp.bfloat16)
        xBs[nsl, :] = xb.astype(jnp.bfloat16)
        heA = jnp.dot(At, xa, preferred_element_type=jnp.float32)
        heB = jnp.dot(Bt, xb, preferred_element_type=jnp.float32)

        @pl.when(nt == 0)
        def _():
            heA_s[...] = heA
            heB_s[...] = heB

        @pl.when(nt != 0)
        def _():
            heA_s[...] += heA
            heB_s[...] += heB

    @pl.when(ph == 1)
    def _():
        @pl.when(nt == 0)
        def _():
            heAt = _gelu_exact(
                jnp.dot(heA_s[...], wb2a_ref[...],
                        preferred_element_type=jnp.float32) + bb2a_ref[...])
            heBt = _gelu_exact(
                jnp.dot(heB_s[...], wa2b_ref[...],
                        preferred_element_type=jnp.float32) + ba2b_ref[...])
            heAt_s[...] = heAt
            heBt_s[...] = heBt
            mA_s[...] = jnp.dot(heBt, wgA_ref[_D:, :],
                                preferred_element_type=jnp.float32)
            mB_s[...] = jnp.dot(heAt, wgB_ref[_D:, :],
                                preferred_element_type=jnp.float32)

        At = AtS[:, nsl]
        Bt = BtS[:, nsl]

        xa = xAs[nsl, :]
        preA = (jnp.dot(xa, wgAtop_s[...], preferred_element_type=jnp.float32)
                + jax.lax.dot_general(At, mA_s[...], _DN0,
                                      preferred_element_type=jnp.float32)
                + bgA_ref[...])
        gA = jax.nn.sigmoid(preA)
        xAfromB = jax.lax.dot_general(At, heBt_s[...], _DN0,
                                      preferred_element_type=jnp.float32)
        outA_ref[0] = gA * xa.astype(jnp.float32) + (1.0 - gA) * xAfromB

        xb = xBs[nsl, :]
        preB = (jnp.dot(xb, wgBtop_s[...], preferred_element_type=jnp.float32)
                + jax.lax.dot_general(Bt, mB_s[...], _DN0,
                                      preferred_element_type=jnp.float32)
                + bgB_ref[...])
        gB = jax.nn.sigmoid(preB)
        xBfromA = jax.lax.dot_general(Bt, heAt_s[...], _DN0,
                                      preferred_element_type=jnp.float32)
        outB_ref[0] = gB * xb.astype(jnp.float32) + (1.0 - gB) * xBfromA


def kernel(X_A, X_B, idxA, wA, idxB, wB, E, W_A2B, b_A2B, W_B2A, b_B2A,
           Wg_A, bg_A, Wg_B, bg_B):
    del E
    f32 = jnp.float32
    bf16 = jnp.bfloat16
    last = _NNT - 1

    idxAT = jnp.swapaxes(idxA, 1, 2)
    wAT = jnp.swapaxes(wA, 1, 2)
    idxBT = jnp.swapaxes(idxB, 1, 2)
    wBT = jnp.swapaxes(wB, 1, 2)

    idxt_spec = pl.BlockSpec((1, _KE, _NT),
                             lambda b, ph, n: (b, 0, n + (last - n) * ph))
    x_spec = pl.BlockSpec((1, _NT, _D),
                          lambda b, ph, n: (b, n + (last - n) * ph, 0))
    out_spec = pl.BlockSpec((1, _NT, _D), lambda b, ph, n: (b, n * ph, 0))
    w_spec = pl.BlockSpec((_D, _D), lambda b, ph, n: (0, 0))
    wg_spec = pl.BlockSpec((2 * _D, _D), lambda b, ph, n: (0, 0))
    bias_spec = pl.BlockSpec((1, _D), lambda b, ph, n: (0, 0))

    scr_e = pltpu.VMEM((_E, _D), f32)

    out_A, out_B = pl.pallas_call(
        _body,
        grid=(_B, 2, _NNT),
        in_specs=[idxt_spec, idxt_spec, idxt_spec, idxt_spec, x_spec, x_spec,
                  w_spec, bias_spec, w_spec, bias_spec,
                  wg_spec, wg_spec, bias_spec, bias_spec],
        out_specs=[out_spec, out_spec],
        out_shape=[jax.ShapeDtypeStruct((_B, _N, _D), f32),
                   jax.ShapeDtypeStruct((_B, _N, _D), f32)],
        scratch_shapes=[pltpu.VMEM((_N, _D), bf16), pltpu.VMEM((_N, _D), bf16),
                        pltpu.VMEM((_E, _N), f32), pltpu.VMEM((_E, _N), f32),
                        scr_e, scr_e, scr_e, scr_e, scr_e, scr_e,
                        pltpu.VMEM((_D, _D), bf16), pltpu.VMEM((_D, _D), bf16)],
    )(idxAT, wAT, idxBT, wBT, X_A, X_B,
      W_B2A, b_B2A.reshape(1, _D), W_A2B, b_A2B.reshape(1, _D),
      Wg_A, Wg_B, bg_A.reshape(1, _D), bg_B.reshape(1, _D))

    return (out_A, out_B)
